# Initial kernel scaffold; baseline (speedup 1.0000x reference)
import jax
import jax.numpy as jnp
from jax.experimental import pallas as pl


def kernel(pred, target):
    raise NotImplementedError("write your pallas kernel here")


# speedup vs baseline: 1925.7023x; 1925.7023x over previous
import jax
import jax.numpy as jnp
from jax.experimental import pallas as pl
from jax.experimental.pallas import tpu as pltpu

_BATCH = 262144
_CLASS_NUM = 40
_N = _BATCH * _CLASS_NUM
_COLS = 1024
_ROWS = _N // _COLS
_BLOCK_ROWS = 1024


def _bce_sum_kernel(p_ref, t_ref, out_ref):
    i = pl.program_id(0)
    p = p_ref[...]
    t = t_ref[...]
    term = jnp.maximum(p, 0.0) - p * t + jnp.log1p(jnp.exp(-jnp.abs(p)))
    s = jnp.sum(term)

    @pl.when(i == 0)
    def _init():
        out_ref[0] = 0.0

    out_ref[0] += s

    @pl.when(i == pl.num_programs(0) - 1)
    def _finalize():
        out_ref[0] = out_ref[0] / _N


def kernel(pred, target):
    p = pred.reshape(_ROWS, _COLS)
    t = target.reshape(_ROWS, _COLS)
    grid = _ROWS // _BLOCK_ROWS
    out = pl.pallas_call(
        _bce_sum_kernel,
        grid=(grid,),
        in_specs=[
            pl.BlockSpec((_BLOCK_ROWS, _COLS), lambda i: (i, 0)),
            pl.BlockSpec((_BLOCK_ROWS, _COLS), lambda i: (i, 0)),
        ],
        out_specs=pl.BlockSpec(
            (1,), lambda i: (0,), memory_space=pltpu.SMEM
        ),
        out_shape=jax.ShapeDtypeStruct((1,), jnp.float32),
    )(p, t)
    return out[0]


# speedup vs baseline: 2559.8823x; 1.3293x over previous
import jax
import jax.numpy as jnp
from jax.experimental import pallas as pl
from jax.experimental.pallas import tpu as pltpu

_BATCH = 262144
_CLASS_NUM = 40
_N = _BATCH * _CLASS_NUM
_COLS = 1024
_ROWS = _N // _COLS
_BLOCK_ROWS = 1024


def _bce_sum_kernel(p_ref, t_ref, out_ref):
    i = pl.program_id(0)
    p = p_ref[...]
    t = t_ref[...]
    term = jnp.maximum(p, 0.0) - p * t + jnp.log1p(jnp.exp(-jnp.abs(p)))
    s = jnp.sum(term)

    @pl.when(i == 0)
    def _init():
        out_ref[0] = 0.0

    out_ref[0] += s

    @pl.when(i == pl.num_programs(0) - 1)
    def _finalize():
        out_ref[0] = out_ref[0] / _N


_BLOCK_B = 16384


def kernel(pred, target):
    grid = _BATCH // _BLOCK_B
    out = pl.pallas_call(
        _bce_sum_kernel,
        grid=(grid,),
        in_specs=[
            pl.BlockSpec((_BLOCK_B, _CLASS_NUM), lambda i: (i, 0)),
            pl.BlockSpec((_BLOCK_B, _CLASS_NUM), lambda i: (i, 0)),
        ],
        out_specs=pl.BlockSpec(
            (1,), lambda i: (0,), memory_space=pltpu.SMEM
        ),
        out_shape=jax.ShapeDtypeStruct((1,), jnp.float32),
    )(pred, target)
    return out[0]


# speedup vs baseline: 2580.9059x; 1.0082x over previous
import jax
import jax.numpy as jnp
from jax.experimental import pallas as pl
from jax.experimental.pallas import tpu as pltpu

_BATCH = 262144
_CLASS_NUM = 40
_N = _BATCH * _CLASS_NUM
_COLS = 1024
_ROWS = _N // _COLS
_BLOCK_ROWS = 1024


def _bce_sum_kernel(p_ref, t_ref, out_ref):
    i = pl.program_id(0)
    p = p_ref[...]
    t = t_ref[...]
    term = jnp.maximum(p, 0.0) - p * t + jnp.log1p(jnp.exp(-jnp.abs(p)))
    s = jnp.sum(term)

    @pl.when(i == 0)
    def _init():
        out_ref[0] = 0.0

    out_ref[0] += s

    @pl.when(i == pl.num_programs(0) - 1)
    def _finalize():
        out_ref[0] = out_ref[0] / _N


_BLOCK_B = 8192


def kernel(pred, target):
    grid = _BATCH // _BLOCK_B
    out = pl.pallas_call(
        _bce_sum_kernel,
        grid=(grid,),
        in_specs=[
            pl.BlockSpec((_BLOCK_B, _CLASS_NUM), lambda i: (i, 0)),
            pl.BlockSpec((_BLOCK_B, _CLASS_NUM), lambda i: (i, 0)),
        ],
        out_specs=pl.BlockSpec(
            (1,), lambda i: (0,), memory_space=pltpu.SMEM
        ),
        out_shape=jax.ShapeDtypeStruct((1,), jnp.float32),
    )(pred, target)
    return out[0]


# speedup vs baseline: 2612.6696x; 1.0123x over previous
import jax
import jax.numpy as jnp
from jax.experimental import pallas as pl
from jax.experimental.pallas import tpu as pltpu

_BATCH = 262144
_CLASS_NUM = 40
_N = _BATCH * _CLASS_NUM
_BLOCK_B = 8192


def _bce_sum_kernel(p_ref, t_ref, out_ref, acc_ref):
    i = pl.program_id(0)

    @pl.when(i == 0)
    def _init():
        acc_ref[...] = jnp.zeros_like(acc_ref)

    p = p_ref[...]
    t = t_ref[...]
    term = jnp.maximum(p, 0.0) - p * t + jnp.log1p(jnp.exp(-jnp.abs(p)))
    acc_ref[...] += jnp.sum(
        term.reshape(_BLOCK_B // 8, 8, _CLASS_NUM), axis=0
    )

    @pl.when(i == pl.num_programs(0) - 1)
    def _finalize():
        out_ref[0] = jnp.sum(acc_ref[...]) / _N


def kernel(pred, target):
    grid = _BATCH // _BLOCK_B
    out = pl.pallas_call(
        _bce_sum_kernel,
        grid=(grid,),
        in_specs=[
            pl.BlockSpec((_BLOCK_B, _CLASS_NUM), lambda i: (i, 0)),
            pl.BlockSpec((_BLOCK_B, _CLASS_NUM), lambda i: (i, 0)),
        ],
        out_specs=pl.BlockSpec(
            (1,), lambda i: (0,), memory_space=pltpu.SMEM
        ),
        out_shape=jax.ShapeDtypeStruct((1,), jnp.float32),
        scratch_shapes=[pltpu.VMEM((8, _CLASS_NUM), jnp.float32)],
    )(pred, target)
    return out[0]


# speedup vs baseline: 2818.0393x; 1.0786x over previous
import jax
import jax.numpy as jnp
from jax.experimental import pallas as pl
from jax.experimental.pallas import tpu as pltpu

_BATCH = 262144
_CLASS_NUM = 40
_N = _BATCH * _CLASS_NUM
_BLOCK_B = 8192


_CHUNK = 256


def _bce_sum_kernel(p_ref, t_ref, out_ref, acc_ref):
    i = pl.program_id(0)

    @pl.when(i == 0)
    def _init():
        acc_ref[...] = jnp.zeros_like(acc_ref)

    def body(j, acc):
        p = p_ref[pl.ds(j * _CHUNK, _CHUNK), :]
        t = t_ref[pl.ds(j * _CHUNK, _CHUNK), :]
        term = jnp.maximum(p, 0.0) - p * t + jnp.log1p(jnp.exp(-jnp.abs(p)))
        return acc + jnp.sum(term.reshape(_CHUNK // 8, 8, _CLASS_NUM), axis=0)

    acc_ref[...] += jax.lax.fori_loop(
        0, _BLOCK_B // _CHUNK, body,
        jnp.zeros((8, _CLASS_NUM), jnp.float32),
    )

    @pl.when(i == pl.num_programs(0) - 1)
    def _finalize():
        out_ref[0] = jnp.sum(acc_ref[...]) / _N


def kernel(pred, target):
    grid = _BATCH // _BLOCK_B
    out = pl.pallas_call(
        _bce_sum_kernel,
        grid=(grid,),
        in_specs=[
            pl.BlockSpec((_BLOCK_B, _CLASS_NUM), lambda i: (i, 0)),
            pl.BlockSpec((_BLOCK_B, _CLASS_NUM), lambda i: (i, 0)),
        ],
        out_specs=pl.BlockSpec(
            (1,), lambda i: (0,), memory_space=pltpu.SMEM
        ),
        out_shape=jax.ShapeDtypeStruct((1,), jnp.float32),
        scratch_shapes=[pltpu.VMEM((8, _CLASS_NUM), jnp.float32)],
    )(pred, target)
    return out[0]


# speedup vs baseline: 2861.1776x; 1.0153x over previous
import jax
import jax.numpy as jnp
from jax.experimental import pallas as pl
from jax.experimental.pallas import tpu as pltpu

_BATCH = 262144
_CLASS_NUM = 40
_N = _BATCH * _CLASS_NUM
_BLOCK_B = 8192
_HALF_BLOCKS = (_BATCH // 2) // _BLOCK_B
_CHUNK = 256


def _bce_term_sum(p_ref, t_ref, acc):
    def body(j, acc):
        p = p_ref[pl.ds(j * _CHUNK, _CHUNK), :]
        t = t_ref[pl.ds(j * _CHUNK, _CHUNK), :]
        term = jnp.maximum(p, 0.0) - p * t + jnp.log1p(jnp.exp(-jnp.abs(p)))
        return acc + jnp.sum(term.reshape(_CHUNK // 8, 8, _CLASS_NUM), axis=0)

    return jax.lax.fori_loop(0, _BLOCK_B // _CHUNK, body, acc)


def _bce_sum_kernel(p0_ref, t0_ref, p1_ref, t1_ref, out_ref, acc_ref):
    i = pl.program_id(0)

    @pl.when(i == 0)
    def _init():
        acc_ref[...] = jnp.zeros_like(acc_ref)

    acc = jnp.zeros((8, _CLASS_NUM), jnp.float32)
    acc = _bce_term_sum(p0_ref, t0_ref, acc)
    acc = _bce_term_sum(p1_ref, t1_ref, acc)
    acc_ref[...] += acc

    @pl.when(i == pl.num_programs(0) - 1)
    def _finalize():
        out_ref[0] = jnp.sum(acc_ref[...]) / _N


def kernel(pred, target):
    lo = pl.BlockSpec((_BLOCK_B, _CLASS_NUM), lambda i: (i, 0))
    hi = pl.BlockSpec((_BLOCK_B, _CLASS_NUM), lambda i: (i + _HALF_BLOCKS, 0))
    out = pl.pallas_call(
        _bce_sum_kernel,
        grid=(_HALF_BLOCKS,),
        in_specs=[lo, lo, hi, hi],
        out_specs=pl.BlockSpec(
            (1,), lambda i: (0,), memory_space=pltpu.SMEM
        ),
        out_shape=jax.ShapeDtypeStruct((1,), jnp.float32),
        scratch_shapes=[pltpu.VMEM((8, _CLASS_NUM), jnp.float32)],
    )(pred, target, pred, target)
    return out[0]


# speedup vs baseline: 2977.8244x; 1.0408x over previous
import jax
import jax.numpy as jnp
from jax.experimental import pallas as pl
from jax.experimental.pallas import tpu as pltpu

_BATCH = 262144
_CLASS_NUM = 40
_N = _BATCH * _CLASS_NUM
_BLOCK_B = 8192
_HALF_BLOCKS = (_BATCH // 2) // _BLOCK_B
_CHUNK = 128


def _bce_term_sum(p_ref, t_ref, acc):
    def body(j, acc):
        p = p_ref[pl.ds(j * _CHUNK, _CHUNK), :]
        t = t_ref[pl.ds(j * _CHUNK, _CHUNK), :]
        a = jnp.abs(p)
        u = jnp.exp2(-1.4426950408889634 * a)
        term = 0.5 * (p + a) - p * t + 0.6931471805599453 * jnp.log2(1.0 + u)
        return acc + jnp.sum(term.reshape(_CHUNK // 8, 8, _CLASS_NUM), axis=0)

    return jax.lax.fori_loop(0, _BLOCK_B // _CHUNK, body, acc)


def _bce_sum_kernel(p0_ref, t0_ref, p1_ref, t1_ref, out_ref, acc_ref):
    i = pl.program_id(0)

    @pl.when(i == 0)
    def _init():
        acc_ref[...] = jnp.zeros_like(acc_ref)

    acc = jnp.zeros((8, _CLASS_NUM), jnp.float32)
    acc = _bce_term_sum(p0_ref, t0_ref, acc)
    acc = _bce_term_sum(p1_ref, t1_ref, acc)
    acc_ref[...] += acc

    @pl.when(i == pl.num_programs(0) - 1)
    def _finalize():
        out_ref[0] = jnp.sum(acc_ref[...]) / _N


def kernel(pred, target):
    lo = pl.BlockSpec((_BLOCK_B, _CLASS_NUM), lambda i: (i, 0))
    hi = pl.BlockSpec((_BLOCK_B, _CLASS_NUM), lambda i: (i + _HALF_BLOCKS, 0))
    out = pl.pallas_call(
        _bce_sum_kernel,
        grid=(_HALF_BLOCKS,),
        in_specs=[lo, lo, hi, hi],
        out_specs=pl.BlockSpec(
            (1,), lambda i: (0,), memory_space=pltpu.SMEM
        ),
        out_shape=jax.ShapeDtypeStruct((1,), jnp.float32),
        scratch_shapes=[pltpu.VMEM((8, _CLASS_NUM), jnp.float32)],
    )(pred, target, pred, target)
    return out[0]
